# baseline (speedup 1.0000x reference)
import functools

import jax
import jax.numpy as jnp
from jax import lax
from jax.experimental import pallas as pl
from jax.experimental.pallas import tpu as pltpu
from jax.experimental.pallas import tpu_sc as plsc

N = 100000
K = 32
NC, NS, L = 2, 16, 16
NW = NC * NS
P = 128
TOTAL_STEPS = -(-N // P)
STEPS = -(-TOTAL_STEPS // NW)
LAST_BASE = N - P
PACK_CHUNKS = -(-TOTAL_STEPS // NS)


def _sc_body(xyz_hbm, int_hbm, idx_hbm, params_hbm, out_hbm,
             idx_v, rows_v, centers_v, idx_v2, rows_v2, centers_v2,
             out_v, params_v, xyz_b, int_b, pack_b, table_sh, semA, semB):
    wid = lax.axis_index("s") * NC + lax.axis_index("c")
    sid = lax.axis_index("s")

    zero16 = jnp.zeros((L,), jnp.int32)
    iota = lax.broadcasted_iota(jnp.int32, (L,), 0)
    c0 = zero16
    c1 = zero16 + 1
    c2 = zero16 + 2
    c3 = zero16 + 3
    c4 = zero16 + 4
    c5 = zero16 + 5
    zf = jnp.zeros((L,), jnp.float32)


    def pack_chunk(ch, carry):
        cid = jnp.minimum(sid * PACK_CHUNKS + ch, TOTAL_STEPS - 1)
        r0 = jnp.minimum(cid * P, N - P)
        pltpu.sync_copy(xyz_hbm.at[pl.ds(r0, P)], xyz_b)
        pltpu.sync_copy(int_hbm.at[:, pl.ds(r0, P)], int_b)
        for g in range(P // L):
            m = g * L + iota
            for c, cq in ((c0, c0), (c1, c1), (c2, c2)):
                t = plsc.load_gather(xyz_b, [m, c])
                plsc.store_scatter(pack_b, [m, cq], t)
            for c, cq in ((c0, c3), (c1, c4), (c2, c5)):
                t = plsc.load_gather(int_b, [c, m])
                plsc.store_scatter(pack_b, [m, cq], t)
        pltpu.sync_copy(pack_b, table_sh.at[pl.ds(r0, P)])
        return carry

    lax.fori_loop(0, PACK_CHUNKS, pack_chunk, 0)

    @pl.when(sid == 0)
    def _():
        for c in range(6):
            plsc.store_scatter(pack_b, [iota, zero16 + c], zf)
        pltpu.sync_copy(pack_b.at[pl.ds(0, L)], table_sh.at[pl.ds(N, L)])

    plsc.subcore_barrier()

    pltpu.sync_copy(params_hbm, params_v)
    v0 = plsc.load_gather(params_v, [zero16, iota])
    v1 = plsc.load_gather(params_v, [zero16 + 1, iota])
    v2 = plsc.load_gather(params_v, [zero16 + 2, iota])
    cc = plsc.load_gather(params_v, [zero16 + 3, iota])

    def sbase(t):
        s = wid * STEPS + jnp.minimum(t, STEPS - 1)
        return jnp.minimum(s * P, LAST_BASE)

    def fire(t, idx_b, rows_b, cent_b, sem_b):
        base = sbase(t)
        pltpu.sync_copy(idx_hbm.at[pl.ds(base * K, P * K)], idx_b)
        pltpu.async_copy(table_sh.at[idx_b], rows_b, sem_b)
        pltpu.async_copy(table_sh.at[pl.ds(base, P)], cent_b, sem_b)

    def drain(idx_b, rows_b, cent_b, sem_b):
        pltpu.make_async_copy(table_sh.at[idx_b], rows_b, sem_b).wait()
        pltpu.make_async_copy(table_sh.at[pl.ds(0, P)], cent_b, sem_b).wait()

    def compute(t, rows_b, cent_b):
        base = sbase(t)
        for g in range(P // L):
            m_pts = g * L + iota
            cx = plsc.load_gather(cent_b, [m_pts, c0])
            cy = plsc.load_gather(cent_b, [m_pts, c1])
            cz = plsc.load_gather(cent_b, [m_pts, c2])
            qb = m_pts * K

            def nbody(k, acc):
                d, a0, a1, a2 = acc
                q = qb + k
                x = plsc.load_gather(rows_b, [q, c0])
                y = plsc.load_gather(rows_b, [q, c1])
                z = plsc.load_gather(rows_b, [q, c2])
                dx = x - cx
                dy = y - cy
                dz = z - cz
                e0 = jnp.exp(dx * dx * -2.0)
                e1 = jnp.exp(dy * dy * -2.0)
                e2 = jnp.exp(dz * dz * -2.0)
                w = v0 * e0 + v1 * e1 + v2 * e2 - cc
                e = jnp.exp(w)
                i0 = plsc.load_gather(rows_b, [q, c3])
                i1 = plsc.load_gather(rows_b, [q, c4])
                i2 = plsc.load_gather(rows_b, [q, c5])
                return d + e, a0 + e * i0, a1 + e * i1, a2 + e * i2

            d, a0, a1, a2 = lax.fori_loop(0, K, nbody, (zf, zf, zf, zf))
            plsc.store_scatter(out_v, [c0, m_pts], a0 / d)
            plsc.store_scatter(out_v, [c1, m_pts], a1 / d)
            plsc.store_scatter(out_v, [c2, m_pts], a2 / d)

        pltpu.sync_copy(out_v, out_hbm.at[:, pl.ds(base, P)])

    bufA = (idx_v, rows_v, centers_v, semA)
    bufB = (idx_v2, rows_v2, centers_v2, semB)
    fire(0, *bufA)

    def dstep(j, carry):
        t = j * 2
        fire(t + 1, *bufB)
        drain(*bufA)
        compute(t, rows_v, centers_v)
        fire(t + 2, *bufA)
        drain(*bufB)
        compute(t + 1, rows_v2, centers_v2)
        return carry

    lax.fori_loop(0, STEPS // 2, dstep, 0)
    drain(*bufA)
    compute(STEPS - 1, rows_v, centers_v)


@jax.jit
def _spatial_attention_sc(xyz1, inten, idxf, params):
    mesh = plsc.VectorSubcoreMesh(core_axis_name="c", subcore_axis_name="s",
                                  num_cores=NC, num_subcores=NS)
    f = pl.kernel(
        _sc_body,
        out_type=jax.ShapeDtypeStruct((3, N), jnp.float32),
        mesh=mesh,
        scratch_types=[
            pltpu.VMEM((P * K,), jnp.int32),
            pltpu.VMEM((P * K, 6), jnp.float32),
            pltpu.VMEM((P, 6), jnp.float32),
            pltpu.VMEM((P * K,), jnp.int32),
            pltpu.VMEM((P * K, 6), jnp.float32),
            pltpu.VMEM((P, 6), jnp.float32),
            pltpu.VMEM((3, P), jnp.float32),
            pltpu.VMEM((4, 128), jnp.float32),
            pltpu.VMEM((P, 3), jnp.float32),
            pltpu.VMEM((3, P), jnp.float32),
            pltpu.VMEM((P, 6), jnp.float32),
            pltpu.VMEM_SHARED((N + 16, 6), jnp.float32),
            pltpu.SemaphoreType.DMA,
            pltpu.SemaphoreType.DMA,
        ],
        compiler_params=pltpu.CompilerParams(needs_layout_passes=False,
                                             use_tc_tiling_on_sc=False),
    )
    return f(xyz1, inten, idxf, params)


def kernel(xyz, intensity, indices, W1, b1, gamma, beta, W2, b2):
    scale = gamma / jnp.sqrt(1.0 + 1e-5)
    v = (W2[0] * scale) @ W1
    shift = jnp.sum(jnp.maximum(v, 0.0))
    pvec = jnp.concatenate([v, shift[None]]).astype(jnp.float32)
    params = jnp.tile(pvec[:, None], (1, 128))

    idxf = indices[0].astype(jnp.int32).reshape(N * K)
    res = _spatial_attention_sc(xyz[0], intensity[0], idxf, params)
    return res[None]


# speedup vs baseline: 1.0023x; 1.0023x over previous
import functools

import jax
import jax.numpy as jnp
from jax import lax
from jax.experimental import pallas as pl
from jax.experimental.pallas import tpu as pltpu
from jax.experimental.pallas import tpu_sc as plsc

N = 100000
K = 32
NC, NS, L = 2, 16, 16
NW = NC * NS
P = 128
TOTAL_STEPS = -(-N // P)
STEPS = -(-TOTAL_STEPS // NW)
LAST_BASE = N - P
PACK_CHUNKS = -(-TOTAL_STEPS // NS)


def _sc_body(xyz_hbm, int_hbm, idx_hbm, params_hbm, out_hbm,
             idx_v, rows_v, centers_v, idx_v2, rows_v2, centers_v2,
             out_v, params_v, xyz_b, int_b, pack_b, table_sh, semA, semB):
    wid = lax.axis_index("s") * NC + lax.axis_index("c")
    sid = lax.axis_index("s")

    zero16 = jnp.zeros((L,), jnp.int32)
    iota = lax.broadcasted_iota(jnp.int32, (L,), 0)
    c0 = zero16
    c1 = zero16 + 1
    c2 = zero16 + 2
    c3 = zero16 + 3
    c4 = zero16 + 4
    c5 = zero16 + 5
    zf = jnp.zeros((L,), jnp.float32)


    def pack_chunk(ch, carry):
        cid = jnp.minimum(sid * PACK_CHUNKS + ch, TOTAL_STEPS - 1)
        r0 = jnp.minimum(cid * P, N - P)
        pltpu.sync_copy(xyz_hbm.at[pl.ds(r0, P)], xyz_b)
        pltpu.sync_copy(int_hbm.at[:, pl.ds(r0, P)], int_b)
        for g in range(P // L):
            m = g * L + iota
            for c, cq in ((c0, c0), (c1, c1), (c2, c2)):
                t = plsc.load_gather(xyz_b, [m, c])
                plsc.store_scatter(pack_b, [m, cq], t)
            for c, cq in ((c0, c3), (c1, c4), (c2, c5)):
                t = plsc.load_gather(int_b, [c, m])
                plsc.store_scatter(pack_b, [m, cq], t)
        pltpu.sync_copy(pack_b, table_sh.at[pl.ds(r0, P)])
        return carry

    lax.fori_loop(0, PACK_CHUNKS, pack_chunk, 0)

    @pl.when(sid == 0)
    def _():
        for c in range(6):
            plsc.store_scatter(pack_b, [iota, zero16 + c], zf)
        pltpu.sync_copy(pack_b.at[pl.ds(0, L)], table_sh.at[pl.ds(N, L)])

    plsc.subcore_barrier()

    pltpu.sync_copy(params_hbm, params_v)
    v0 = plsc.load_gather(params_v, [zero16, iota])
    v1 = plsc.load_gather(params_v, [zero16 + 1, iota])
    v2 = plsc.load_gather(params_v, [zero16 + 2, iota])
    cc = plsc.load_gather(params_v, [zero16 + 3, iota])

    def sbase(t):
        s = wid * STEPS + jnp.minimum(t, STEPS - 1)
        return jnp.minimum(s * P, LAST_BASE)

    def fire(t, idx_b, rows_b, cent_b, sem_b):
        base = sbase(t)
        pltpu.sync_copy(idx_hbm.at[pl.ds(base * K, P * K)], idx_b)
        pltpu.async_copy(table_sh.at[idx_b], rows_b, sem_b)
        pltpu.async_copy(table_sh.at[pl.ds(base, P)], cent_b, sem_b)

    def drain(idx_b, rows_b, cent_b, sem_b):
        pltpu.make_async_copy(table_sh.at[idx_b], rows_b, sem_b).wait()
        pltpu.make_async_copy(table_sh.at[pl.ds(0, P)], cent_b, sem_b).wait()

    def compute(t, rows_b, cent_b):
        base = sbase(t)
        for g in range(P // L):
            m_pts = g * L + iota
            cx = plsc.load_gather(cent_b, [m_pts, c0])
            cy = plsc.load_gather(cent_b, [m_pts, c1])
            cz = plsc.load_gather(cent_b, [m_pts, c2])
            qb = m_pts * K

            def nbody(k, acc):
                d, a0, a1, a2 = acc
                q = qb + k
                x = plsc.load_gather(rows_b, [q, c0])
                y = plsc.load_gather(rows_b, [q, c1])
                z = plsc.load_gather(rows_b, [q, c2])
                dx = x - cx
                dy = y - cy
                dz = z - cz
                e0 = jnp.exp(dx * dx * -2.0)
                e1 = jnp.exp(dy * dy * -2.0)
                e2 = jnp.exp(dz * dz * -2.0)
                w = v0 * e0 + v1 * e1 + v2 * e2 - cc
                e = jnp.exp(w)
                i0 = plsc.load_gather(rows_b, [q, c3])
                i1 = plsc.load_gather(rows_b, [q, c4])
                i2 = plsc.load_gather(rows_b, [q, c5])
                return d + e, a0 + e * i0, a1 + e * i1, a2 + e * i2

            d, a0, a1, a2 = lax.fori_loop(0, K, nbody, (zf, zf, zf, zf))
            plsc.store_scatter(out_v, [c0, m_pts], a0 / d)
            plsc.store_scatter(out_v, [c1, m_pts], a1 / d)
            plsc.store_scatter(out_v, [c2, m_pts], a2 / d)

        pltpu.sync_copy(out_v, out_hbm.at[:, pl.ds(base, P)])

    bufA = (idx_v, rows_v, centers_v, semA)
    bufB = (idx_v2, rows_v2, centers_v2, semB)
    fire(0, *bufA)

    def dstep(j, carry):
        t = j * 2
        fire(t + 1, *bufB)
        drain(*bufA)
        compute(t, rows_v, centers_v)
        fire(t + 2, *bufA)
        drain(*bufB)
        compute(t + 1, rows_v2, centers_v2)
        return carry

    lax.fori_loop(0, STEPS // 2, dstep, 0)
    drain(*bufA)
    compute(STEPS - 1, rows_v, centers_v)


@jax.jit
def _spatial_attention_sc(xyz1, inten, idxf, params):
    mesh = plsc.VectorSubcoreMesh(core_axis_name="c", subcore_axis_name="s",
                                  num_cores=NC, num_subcores=NS)
    f = pl.kernel(
        _sc_body,
        out_type=jax.ShapeDtypeStruct((3, N), jnp.float32),
        mesh=mesh,
        scratch_types=[
            pltpu.VMEM((P * K,), jnp.int32),
            pltpu.VMEM((P * K, 8), jnp.float32),
            pltpu.VMEM((P, 8), jnp.float32),
            pltpu.VMEM((P * K,), jnp.int32),
            pltpu.VMEM((P * K, 8), jnp.float32),
            pltpu.VMEM((P, 8), jnp.float32),
            pltpu.VMEM((3, P), jnp.float32),
            pltpu.VMEM((4, 128), jnp.float32),
            pltpu.VMEM((P, 3), jnp.float32),
            pltpu.VMEM((3, P), jnp.float32),
            pltpu.VMEM((P, 8), jnp.float32),
            pltpu.VMEM_SHARED((N + 16, 8), jnp.float32),
            pltpu.SemaphoreType.DMA,
            pltpu.SemaphoreType.DMA,
        ],
        compiler_params=pltpu.CompilerParams(needs_layout_passes=False,
                                             use_tc_tiling_on_sc=False),
    )
    return f(xyz1, inten, idxf, params)


def kernel(xyz, intensity, indices, W1, b1, gamma, beta, W2, b2):
    scale = gamma / jnp.sqrt(1.0 + 1e-5)
    v = (W2[0] * scale) @ W1
    shift = jnp.sum(jnp.maximum(v, 0.0))
    pvec = jnp.concatenate([v, shift[None]]).astype(jnp.float32)
    params = jnp.tile(pvec[:, None], (1, 128))

    idxf = indices[0].astype(jnp.int32).reshape(N * K)
    res = _spatial_attention_sc(xyz[0], intensity[0], idxf, params)
    return res[None]


# speedup vs baseline: 1.0028x; 1.0005x over previous
import jax
import jax.numpy as jnp
from jax import lax
from jax.experimental import pallas as pl
from jax.experimental.pallas import tpu as pltpu
from jax.experimental.pallas import tpu_sc as plsc

N = 100000
K = 32
NC, NS, L = 2, 16, 16
NW = NC * NS
P = 128
TOTAL_STEPS = -(-N // P)
STEPS = -(-TOTAL_STEPS // NW)
LAST_BASE = N - P
PACK_CHUNKS = -(-TOTAL_STEPS // NS)


def _sc_body(xyz_hbm, int_hbm, idx_hbm, params_hbm, out_hbm,
             idx_v, rows_v, centers_v, idx_v2, rows_v2, centers_v2,
             out_v, params_v, xyz_b, int_b, pack_b, table_sh, semA, semB):
    wid = lax.axis_index("s") * NC + lax.axis_index("c")
    sid = lax.axis_index("s")

    zero16 = jnp.zeros((L,), jnp.int32)
    iota = lax.broadcasted_iota(jnp.int32, (L,), 0)
    c0 = zero16
    c1 = zero16 + 1
    c2 = zero16 + 2
    c3 = zero16 + 3
    c4 = zero16 + 4
    c5 = zero16 + 5
    zf = jnp.zeros((L,), jnp.float32)


    def pack_chunk(ch, carry):
        cid = jnp.minimum(sid * PACK_CHUNKS + ch, TOTAL_STEPS - 1)
        r0 = jnp.minimum(cid * P, N - P)
        pltpu.sync_copy(xyz_hbm.at[pl.ds(r0, P)], xyz_b)
        pltpu.sync_copy(int_hbm.at[:, pl.ds(r0, P)], int_b)
        for g in range(P // L):
            m = g * L + iota
            for c, cq in ((c0, c0), (c1, c1), (c2, c2)):
                t = plsc.load_gather(xyz_b, [m, c])
                plsc.store_scatter(pack_b, [m, cq], t)
            for c, cq in ((c0, c3), (c1, c4), (c2, c5)):
                t = plsc.load_gather(int_b, [c, m])
                plsc.store_scatter(pack_b, [m, cq], t)
        pltpu.sync_copy(pack_b, table_sh.at[pl.ds(r0, P)])
        return carry

    lax.fori_loop(0, PACK_CHUNKS, pack_chunk, 0)

    @pl.when(sid == 0)
    def _():
        for c in range(6):
            plsc.store_scatter(pack_b, [iota, zero16 + c], zf)
        pltpu.sync_copy(pack_b.at[pl.ds(0, L)], table_sh.at[pl.ds(N, L)])

    plsc.subcore_barrier()

    pltpu.sync_copy(params_hbm, params_v)
    v0 = plsc.load_gather(params_v, [zero16, iota])
    v1 = plsc.load_gather(params_v, [zero16 + 1, iota])
    v2 = plsc.load_gather(params_v, [zero16 + 2, iota])
    cc = plsc.load_gather(params_v, [zero16 + 3, iota])

    def sbase(t):
        s = wid * STEPS + jnp.minimum(t, STEPS - 1)
        return jnp.minimum(s * P, LAST_BASE)

    def fire(t, idx_b, rows_b, cent_b, sem_b):
        base = sbase(t)
        pltpu.sync_copy(idx_hbm.at[pl.ds(base * K, P * K)], idx_b)
        pltpu.async_copy(table_sh.at[idx_b], rows_b, sem_b)
        pltpu.async_copy(table_sh.at[pl.ds(base, P)], cent_b, sem_b)

    def drain(idx_b, rows_b, cent_b, sem_b):
        pltpu.make_async_copy(table_sh.at[idx_b], rows_b, sem_b).wait()
        pltpu.make_async_copy(table_sh.at[pl.ds(0, P)], cent_b, sem_b).wait()

    def compute(t, rows_b, cent_b):
        base = sbase(t)
        for g in range(P // L):
            m_pts = g * L + iota
            cx = plsc.load_gather(cent_b, [m_pts, c0])
            cy = plsc.load_gather(cent_b, [m_pts, c1])
            cz = plsc.load_gather(cent_b, [m_pts, c2])
            qb = m_pts * K

            def nbody(k, acc):
                d, a0, a1, a2 = acc
                q = qb + k
                x = plsc.load_gather(rows_b, [q, c0])
                y = plsc.load_gather(rows_b, [q, c1])
                z = plsc.load_gather(rows_b, [q, c2])
                dx = x - cx
                dy = y - cy
                dz = z - cz
                e0 = jnp.exp(dx * dx * -2.0)
                e1 = jnp.exp(dy * dy * -2.0)
                e2 = jnp.exp(dz * dz * -2.0)
                w = v0 * e0 + v1 * e1 + v2 * e2 - cc
                e = jnp.exp(w)
                i0 = plsc.load_gather(rows_b, [q, c3])
                i1 = plsc.load_gather(rows_b, [q, c4])
                i2 = plsc.load_gather(rows_b, [q, c5])
                return d + e, a0 + e * i0, a1 + e * i1, a2 + e * i2

            d, a0, a1, a2 = lax.fori_loop(0, K, nbody, (zf, zf, zf, zf))
            plsc.store_scatter(out_v, [c0, m_pts], a0 / d)
            plsc.store_scatter(out_v, [c1, m_pts], a1 / d)
            plsc.store_scatter(out_v, [c2, m_pts], a2 / d)

        pltpu.sync_copy(out_v, out_hbm.at[:, pl.ds(base, P)])

    bufA = (idx_v, rows_v, centers_v, semA)
    bufB = (idx_v2, rows_v2, centers_v2, semB)
    fire(0, *bufA)

    def dstep(j, carry):
        t = j * 2
        fire(t + 1, *bufB)
        drain(*bufA)
        compute(t, rows_v, centers_v)
        fire(t + 2, *bufA)
        drain(*bufB)
        compute(t + 1, rows_v2, centers_v2)
        return carry

    lax.fori_loop(0, STEPS // 2, dstep, 0)
    drain(*bufA)
    compute(STEPS - 1, rows_v, centers_v)


@jax.jit
def _spatial_attention_sc(xyz1, inten, idxf, params):
    mesh = plsc.VectorSubcoreMesh(core_axis_name="c", subcore_axis_name="s",
                                  num_cores=NC, num_subcores=NS)
    f = pl.kernel(
        _sc_body,
        out_type=jax.ShapeDtypeStruct((3, N), jnp.float32),
        mesh=mesh,
        scratch_types=[
            pltpu.VMEM((P * K,), jnp.int32),
            pltpu.VMEM((P * K, 8), jnp.float32),
            pltpu.VMEM((P, 8), jnp.float32),
            pltpu.VMEM((P * K,), jnp.int32),
            pltpu.VMEM((P * K, 8), jnp.float32),
            pltpu.VMEM((P, 8), jnp.float32),
            pltpu.VMEM((3, P), jnp.float32),
            pltpu.VMEM((4, 128), jnp.float32),
            pltpu.VMEM((P, 3), jnp.float32),
            pltpu.VMEM((3, P), jnp.float32),
            pltpu.VMEM((P, 8), jnp.float32),
            pltpu.VMEM_SHARED((N + 16, 8), jnp.float32),
            pltpu.SemaphoreType.DMA,
            pltpu.SemaphoreType.DMA,
        ],
        compiler_params=pltpu.CompilerParams(needs_layout_passes=False,
                                             use_tc_tiling_on_sc=False),
    )
    return f(xyz1, inten, idxf, params)


def kernel(xyz, intensity, indices, W1, b1, gamma, beta, W2, b2):
    scale = gamma / jnp.sqrt(1.0 + 1e-5)
    v = (W2[0] * scale) @ W1
    shift = jnp.sum(jnp.maximum(v, 0.0))
    pvec = jnp.concatenate([v, shift[None]]).astype(jnp.float32)
    params = jnp.tile(pvec[:, None], (1, 128))

    idxf = indices[0].astype(jnp.int32).reshape(N * K)
    res = _spatial_attention_sc(xyz[0], intensity[0], idxf, params)
    return res[None]
